# baseline (speedup 1.0000x reference)
import functools

import jax
import jax.numpy as jnp
from jax import lax
from jax.experimental import pallas as pl
from jax.experimental.pallas import tpu as pltpu
from jax.experimental.pallas import tpu_sc as plsc

E = 8
TOPK = 2
D = 768
F = 768
T = 2048
A = T * TOPK

B = 256
NB = A // B + E
RMAX = NB * B

NW = 32
RPW = RMAX // NW
GCH = 64
TPW = T // NW
CT = 32

_NEG = -1e30


def _router_body(x_ref, wg_ref, pos_ref, wts_ref, end_ref):
    x = x_ref[...]
    logits = jnp.dot(x, wg_ref[...], preferred_element_type=jnp.float32)
    idx = lax.broadcasted_iota(jnp.int32, (T, E), 1)
    m1 = jnp.max(logits, axis=1, keepdims=True)
    i1 = jnp.min(jnp.where(logits == m1, idx, E), axis=1, keepdims=True)
    l2 = jnp.where(idx == i1, _NEG, logits)
    m2 = jnp.max(l2, axis=1, keepdims=True)
    i2 = jnp.min(jnp.where(l2 == m2, idx, E), axis=1, keepdims=True)
    wa = jax.nn.sigmoid(m1 - m2)

    oh1 = (idx == i1).astype(jnp.float32)
    oh2 = (idx == i2).astype(jnp.float32)
    onehot = oh1 + oh2

    CB = 512
    r = lax.broadcasted_iota(jnp.int32, (CB, CB), 0)
    c = lax.broadcasted_iota(jnp.int32, (CB, CB), 1)
    tri = (c < r).astype(jnp.float32)
    run = jnp.zeros((1, E), jnp.float32)
    parts = []
    for bi in range(T // CB):
        ab = onehot[bi * CB:(bi + 1) * CB, :]
        parts.append(jnp.dot(tri, ab, preferred_element_type=jnp.float32) + run)
        run = run + jnp.sum(ab, axis=0, keepdims=True)
    cnt = jnp.concatenate(parts, axis=0)

    seg = jnp.floor((run + (B - 1)) * (1.0 / B)) * B
    er = lax.broadcasted_iota(jnp.int32, (E, E), 0)
    ec = lax.broadcasted_iota(jnp.int32, (E, E), 1)
    tri8 = (er < ec).astype(jnp.float32)
    off = jnp.dot(seg, tri8, preferred_element_type=jnp.float32)

    posc = cnt + off
    p1 = jnp.sum(oh1 * posc, axis=1, keepdims=True)
    p2 = jnp.sum(oh2 * posc, axis=1, keepdims=True)
    pos_ref[:, 0:1] = p1.astype(jnp.int32)
    pos_ref[:, 1:2] = p2.astype(jnp.int32)
    wts_ref[:, 0:1] = wa
    wts_ref[:, 1:2] = 1.0 - wa
    end_ref[...] = (off + seg).astype(jnp.int32)


def _router(x, wg):
    return pl.pallas_call(
        _router_body,
        in_specs=[
            pl.BlockSpec((T, D), lambda: (0, 0)),
            pl.BlockSpec((D, E), lambda: (0, 0)),
        ],
        out_specs=[
            pl.BlockSpec((T, TOPK), lambda: (0, 0)),
            pl.BlockSpec((T, TOPK), lambda: (0, 0)),
            pl.BlockSpec((1, E), lambda: (0, 0)),
        ],
        out_shape=[
            jax.ShapeDtypeStruct((T, TOPK), jnp.int32),
            jax.ShapeDtypeStruct((T, TOPK), jnp.float32),
            jax.ShapeDtypeStruct((1, E), jnp.int32),
        ],
    )(x, wg)


@functools.cache
def _sc_dispatch():
    mesh = plsc.VectorSubcoreMesh(core_axis_name="c", subcore_axis_name="s")
    return functools.partial(
        pl.kernel,
        mesh=mesh,
        compiler_params=pltpu.CompilerParams(needs_layout_passes=False),
        out_type=[
            jax.ShapeDtypeStruct((RMAX, D), jnp.float32),
            jax.ShapeDtypeStruct((RMAX,), jnp.float32),
        ],
        scratch_types=[
            pltpu.VMEM((A,), jnp.int32),
            pltpu.VMEM((A,), jnp.float32),
            pltpu.VMEM((RMAX,), jnp.int32),
            pltpu.VMEM((RMAX,), jnp.float32),
            pltpu.VMEM((GCH, D), jnp.float32),
            pltpu.SemaphoreType.DMA,
        ],
    )(_dispatch_body)


def _dispatch_body(x_hbm, pos_hbm, wts_hbm, xs_hbm, scale_hbm,
                   pos_v, wts_v, r2t_v, scale_v, rows_v, sem):
    wid = lax.axis_index("s") * 2 + lax.axis_index("c")
    pltpu.sync_copy(pos_hbm, pos_v)
    pltpu.sync_copy(wts_hbm, wts_v)

    zi = jnp.zeros((16,), jnp.int32)
    zf = jnp.zeros((16,), jnp.float32)

    def zbody(j, carry):
        r2t_v[pl.ds(j * 16, 16)] = zi
        scale_v[pl.ds(j * 16, 16)] = zf
        return carry

    lax.fori_loop(0, RMAX // 16, zbody, 0)

    lane = lax.iota(jnp.int32, 16)

    def sbody(j, carry):
        idxv = pos_v[pl.ds(j * 16, 16)]
        tok = lax.shift_right_logical(j * 16 + lane, 1)
        plsc.store_scatter(r2t_v, [idxv], tok)
        plsc.store_scatter(scale_v, [idxv], wts_v[pl.ds(j * 16, 16)])
        return carry

    lax.fori_loop(0, A // 16, sbody, 0)

    @pl.when(wid == 0)
    def _():
        pltpu.sync_copy(scale_v, scale_hbm)

    base = wid * RPW
    for ck in range(RPW // GCH):
        idx_ref = r2t_v.at[pl.ds(base + ck * GCH, GCH)]
        pltpu.async_copy(x_hbm.at[idx_ref], rows_v, sem).wait()
        pltpu.sync_copy(rows_v, xs_hbm.at[pl.ds(base + ck * GCH, GCH)])


def _gemm_body(end_ref, xs_ref, scale_ref, w1_ref, w2_ref, ys_ref):
    b = pl.program_id(0)
    nbu = end_ref[E - 1] // B

    @pl.when(b < nbu)
    def _():
        xb = xs_ref[...].astype(jnp.bfloat16)
        w1e = w1_ref[0].astype(jnp.bfloat16)
        h = lax.dot_general(xb, w1e, (((1,), (1,)), ((), ())),
                            preferred_element_type=jnp.float32)
        g = h[:, :F]
        u = h[:, F:]
        act = (g * jax.nn.sigmoid(g) * u).astype(jnp.bfloat16)
        y = jnp.dot(act, w2_ref[0].astype(jnp.bfloat16),
                    preferred_element_type=jnp.float32)
        ys_ref[...] = y * scale_ref[...]


def _row_block(b, end_ref):
    nbu = end_ref[E - 1] // B
    return jnp.minimum(b, nbu - 1)


def _grp(b, end_ref):
    g = jnp.int32(0)
    for e in range(E):
        g = g + (end_ref[e] <= b * B).astype(jnp.int32)
    return jnp.minimum(g, E - 1)


def _gemm(endv, xs, scale, w1, w2):
    grid_spec = pltpu.PrefetchScalarGridSpec(
        num_scalar_prefetch=1,
        grid=(NB,),
        in_specs=[
            pl.BlockSpec((B, D), lambda b, end_ref: (_row_block(b, end_ref), 0)),
            pl.BlockSpec((B, 1), lambda b, end_ref: (_row_block(b, end_ref), 0)),
            pl.BlockSpec((1, 2 * F, D), lambda b, end_ref: (_grp(b, end_ref), 0, 0)),
            pl.BlockSpec((1, F, D), lambda b, end_ref: (_grp(b, end_ref), 0, 0)),
        ],
        out_specs=pl.BlockSpec((B, D), lambda b, end_ref: (_row_block(b, end_ref), 0)),
    )
    return pl.pallas_call(
        _gemm_body,
        grid_spec=grid_spec,
        out_shape=jax.ShapeDtypeStruct((RMAX, D), jnp.float32),
    )(endv, xs, scale, w1, w2)


@functools.cache
def _sc_combine():
    mesh = plsc.VectorSubcoreMesh(core_axis_name="c", subcore_axis_name="s")
    return functools.partial(
        pl.kernel,
        mesh=mesh,
        out_type=jax.ShapeDtypeStruct((T, D), jnp.float32),
        scratch_types=[
            pltpu.VMEM((TOPK * TPW,), jnp.int32),
            pltpu.VMEM((TOPK * CT, D), jnp.float32),
            pltpu.VMEM((CT, D), jnp.float32),
            pltpu.SemaphoreType.DMA,
        ],
    )(_combine_body)


def _combine_body(ys_hbm, pos_hbm, out_hbm, pos_v, buf_v, out_v, sem):
    wid = lax.axis_index("s") * 2 + lax.axis_index("c")
    tbase = wid * TPW
    pltpu.sync_copy(pos_hbm.at[pl.ds(tbase * TOPK, TOPK * TPW)], pos_v)
    for ck in range(TPW // CT):
        idx_ref = pos_v.at[pl.ds(ck * TOPK * CT, TOPK * CT)]
        pltpu.async_copy(ys_hbm.at[idx_ref], buf_v, sem).wait()

        def abody(j, carry):
            def cbody(s, carry2):
                av = buf_v[2 * j, pl.ds(s * 16, 16)]
                bv = buf_v[2 * j + 1, pl.ds(s * 16, 16)]
                out_v[j, pl.ds(s * 16, 16)] = av + bv
                return carry2
            return lax.fori_loop(0, D // 16, cbody, carry)

        lax.fori_loop(0, CT, abody, 0)
        pltpu.sync_copy(out_v, out_hbm.at[pl.ds(tbase + ck * CT, CT)])


@jax.jit
def kernel(hidden_states, Wg, w1, w2):
    pos, wts, endr = _router(hidden_states, Wg)
    posf = pos.reshape(A)
    wtsf = wts.reshape(A)
    endv = endr.reshape(E)
    xs, scale = _sc_dispatch()(hidden_states, posf, wtsf)
    ys = _gemm(endv, xs, scale.reshape(RMAX, 1), w1, w2)
    return _sc_combine()(ys, posf)


# speedup vs baseline: 1.0144x; 1.0144x over previous
import functools

import jax
import jax.numpy as jnp
from jax import lax
from jax.experimental import pallas as pl
from jax.experimental.pallas import tpu as pltpu
from jax.experimental.pallas import tpu_sc as plsc

E = 8
TOPK = 2
D = 768
F = 768
T = 2048
A = T * TOPK

B = 256
NB = A // B + E
RMAX = NB * B

NW = 32
RPW = RMAX // NW
GCH = 64
TPW = T // NW
CT = 32

_NEG = -1e30


def _router_body(x_ref, wg_ref, pos_ref, wts_ref, end_ref):
    x = x_ref[...]
    logits = jnp.dot(x, wg_ref[...], preferred_element_type=jnp.float32)
    idx = lax.broadcasted_iota(jnp.int32, (T, E), 1)
    m1 = jnp.max(logits, axis=1, keepdims=True)
    i1 = jnp.min(jnp.where(logits == m1, idx, E), axis=1, keepdims=True)
    l2 = jnp.where(idx == i1, _NEG, logits)
    m2 = jnp.max(l2, axis=1, keepdims=True)
    i2 = jnp.min(jnp.where(l2 == m2, idx, E), axis=1, keepdims=True)
    wa = jax.nn.sigmoid(m1 - m2)

    oh1 = (idx == i1).astype(jnp.float32)
    oh2 = (idx == i2).astype(jnp.float32)
    onehot = oh1 + oh2

    CB = 512
    r = lax.broadcasted_iota(jnp.int32, (CB, CB), 0)
    c = lax.broadcasted_iota(jnp.int32, (CB, CB), 1)
    tri = (c < r).astype(jnp.float32)
    run = jnp.zeros((1, E), jnp.float32)
    parts = []
    for bi in range(T // CB):
        ab = onehot[bi * CB:(bi + 1) * CB, :]
        parts.append(jnp.dot(tri, ab, preferred_element_type=jnp.float32) + run)
        run = run + jnp.sum(ab, axis=0, keepdims=True)
    cnt = jnp.concatenate(parts, axis=0)

    seg = jnp.floor((run + (B - 1)) * (1.0 / B)) * B
    er = lax.broadcasted_iota(jnp.int32, (E, E), 0)
    ec = lax.broadcasted_iota(jnp.int32, (E, E), 1)
    tri8 = (er < ec).astype(jnp.float32)
    off = jnp.dot(seg, tri8, preferred_element_type=jnp.float32)

    posc = cnt + off
    p1 = jnp.sum(oh1 * posc, axis=1, keepdims=True)
    p2 = jnp.sum(oh2 * posc, axis=1, keepdims=True)
    pos_ref[:, 0:1] = p1.astype(jnp.int32)
    pos_ref[:, 1:2] = p2.astype(jnp.int32)
    wts_ref[:, 0:1] = wa
    wts_ref[:, 1:2] = 1.0 - wa
    end_ref[...] = (off + seg).astype(jnp.int32)


def _router(x, wg):
    return pl.pallas_call(
        _router_body,
        in_specs=[
            pl.BlockSpec((T, D), lambda: (0, 0)),
            pl.BlockSpec((D, E), lambda: (0, 0)),
        ],
        out_specs=[
            pl.BlockSpec((T, TOPK), lambda: (0, 0)),
            pl.BlockSpec((T, TOPK), lambda: (0, 0)),
            pl.BlockSpec((1, E), lambda: (0, 0)),
        ],
        out_shape=[
            jax.ShapeDtypeStruct((T, TOPK), jnp.int32),
            jax.ShapeDtypeStruct((T, TOPK), jnp.float32),
            jax.ShapeDtypeStruct((1, E), jnp.int32),
        ],
    )(x, wg)


APS = A // 16
ZPS = RMAX // 16


@functools.cache
def _sc_dispatch():
    mesh = plsc.VectorSubcoreMesh(core_axis_name="c", subcore_axis_name="s")
    return functools.partial(
        pl.kernel,
        mesh=mesh,
        compiler_params=pltpu.CompilerParams(needs_layout_passes=False),
        out_type=[
            jax.ShapeDtypeStruct((RMAX, D), jnp.float32),
            jax.ShapeDtypeStruct((RMAX,), jnp.float32),
        ],
        scratch_types=[
            pltpu.VMEM((APS,), jnp.int32),
            pltpu.VMEM((APS,), jnp.float32),
            pltpu.VMEM((APS,), jnp.int32),
            pltpu.VMEM((ZPS,), jnp.int32),
            pltpu.VMEM((ZPS,), jnp.float32),
            pltpu.VMEM((RPW,), jnp.int32),
            pltpu.VMEM((GCH, D), jnp.float32),
            pltpu.VMEM((GCH, D), jnp.float32),
            pltpu.VMEM_SHARED((RMAX,), jnp.int32),
            pltpu.VMEM_SHARED((RMAX,), jnp.float32),
            pltpu.SemaphoreType.DMA,
            pltpu.SemaphoreType.DMA,
        ],
    )(_dispatch_body)


def _dispatch_body(x_hbm, pos_hbm, wts_hbm, xs_hbm, scale_hbm,
                   pos_v, wts_v, tok_v, zi_v, zf_v, r2t_v,
                   rows0_v, rows1_v, r2t_sh, scale_sh, sem0, sem1):
    cid = lax.axis_index("c")
    sid = lax.axis_index("s")
    wid = sid * 2 + cid

    abase = sid * APS
    pltpu.sync_copy(pos_hbm.at[pl.ds(abase, APS)], pos_v)
    pltpu.sync_copy(wts_hbm.at[pl.ds(abase, APS)], wts_v)

    zi = jnp.zeros((16,), jnp.int32)
    zf = jnp.zeros((16,), jnp.float32)
    lane = lax.iota(jnp.int32, 16)
    for j in range(ZPS // 16):
        zi_v[pl.ds(j * 16, 16)] = zi
        zf_v[pl.ds(j * 16, 16)] = zf
    for j in range(APS // 16):
        tok_v[pl.ds(j * 16, 16)] = lax.shift_right_logical(
            abase + j * 16 + lane, 1)

    pltpu.sync_copy(zi_v, r2t_sh.at[pl.ds(sid * ZPS, ZPS)])
    pltpu.sync_copy(zf_v, scale_sh.at[pl.ds(sid * ZPS, ZPS)])
    plsc.subcore_barrier()

    pltpu.sync_copy(tok_v, r2t_sh.at[pos_v])
    pltpu.sync_copy(wts_v, scale_sh.at[pos_v])
    plsc.subcore_barrier()

    @pl.when(wid == 0)
    def _():
        pltpu.sync_copy(scale_sh, scale_hbm)

    base = wid * RPW
    pltpu.sync_copy(r2t_sh.at[pl.ds(base, RPW)], r2t_v)
    NCK = RPW // GCH
    bufs = (rows0_v, rows1_v)
    sems = (sem0, sem1)
    handles = [None] * NCK
    for ck in range(min(2, NCK)):
        handles[ck] = pltpu.async_copy(
            x_hbm.at[r2t_v.at[pl.ds(ck * GCH, GCH)]], bufs[ck % 2], sems[ck % 2])
    for ck in range(NCK):
        handles[ck].wait()
        pltpu.sync_copy(bufs[ck % 2], xs_hbm.at[pl.ds(base + ck * GCH, GCH)])
        nxt = ck + 2
        if nxt < NCK:
            handles[nxt] = pltpu.async_copy(
                x_hbm.at[r2t_v.at[pl.ds(nxt * GCH, GCH)]],
                bufs[nxt % 2], sems[nxt % 2])


def _gemm_body(end_ref, xs_ref, scale_ref, w1_ref, w2_ref, ys_ref):
    b = pl.program_id(0)
    nbu = end_ref[E - 1] // B

    @pl.when(b < nbu)
    def _():
        xb = xs_ref[...].astype(jnp.bfloat16)
        w1e = w1_ref[0].astype(jnp.bfloat16)
        h = lax.dot_general(xb, w1e, (((1,), (1,)), ((), ())),
                            preferred_element_type=jnp.float32)
        g = h[:, :F]
        u = h[:, F:]
        act = (g * jax.nn.sigmoid(g) * u).astype(jnp.bfloat16)
        y = jnp.dot(act, w2_ref[0].astype(jnp.bfloat16),
                    preferred_element_type=jnp.float32)
        ys_ref[...] = y * scale_ref[...]


def _row_block(b, end_ref):
    nbu = end_ref[E - 1] // B
    return jnp.minimum(b, nbu - 1)


def _grp(b, end_ref):
    g = jnp.int32(0)
    for e in range(E):
        g = g + (end_ref[e] <= b * B).astype(jnp.int32)
    return jnp.minimum(g, E - 1)


def _gemm(endv, xs, scale, w1, w2):
    grid_spec = pltpu.PrefetchScalarGridSpec(
        num_scalar_prefetch=1,
        grid=(NB,),
        in_specs=[
            pl.BlockSpec((B, D), lambda b, end_ref: (_row_block(b, end_ref), 0)),
            pl.BlockSpec((B, 1), lambda b, end_ref: (_row_block(b, end_ref), 0)),
            pl.BlockSpec((1, 2 * F, D), lambda b, end_ref: (_grp(b, end_ref), 0, 0)),
            pl.BlockSpec((1, F, D), lambda b, end_ref: (_grp(b, end_ref), 0, 0)),
        ],
        out_specs=pl.BlockSpec((B, D), lambda b, end_ref: (_row_block(b, end_ref), 0)),
    )
    return pl.pallas_call(
        _gemm_body,
        grid_spec=grid_spec,
        out_shape=jax.ShapeDtypeStruct((RMAX, D), jnp.float32),
    )(endv, xs, scale, w1, w2)


@functools.cache
def _sc_combine():
    mesh = plsc.VectorSubcoreMesh(core_axis_name="c", subcore_axis_name="s")
    return functools.partial(
        pl.kernel,
        mesh=mesh,
        out_type=jax.ShapeDtypeStruct((T, D), jnp.float32),
        scratch_types=[
            pltpu.VMEM((TOPK * TPW,), jnp.int32),
            pltpu.VMEM((TOPK * CT, D), jnp.float32),
            pltpu.VMEM((CT, D), jnp.float32),
            pltpu.SemaphoreType.DMA,
        ],
    )(_combine_body)


def _combine_body(ys_hbm, pos_hbm, out_hbm, pos_v, buf_v, out_v, sem):
    wid = lax.axis_index("s") * 2 + lax.axis_index("c")
    tbase = wid * TPW
    pltpu.sync_copy(pos_hbm.at[pl.ds(tbase * TOPK, TOPK * TPW)], pos_v)
    for ck in range(TPW // CT):
        idx_ref = pos_v.at[pl.ds(ck * TOPK * CT, TOPK * CT)]
        pltpu.async_copy(ys_hbm.at[idx_ref], buf_v, sem).wait()

        def abody(j, carry):
            for s in range(D // 16):
                av = buf_v[2 * j, pl.ds(s * 16, 16)]
                bv = buf_v[2 * j + 1, pl.ds(s * 16, 16)]
                out_v[j, pl.ds(s * 16, 16)] = av + bv
            return carry

        lax.fori_loop(0, CT, abody, 0)
        pltpu.sync_copy(out_v, out_hbm.at[pl.ds(tbase + ck * CT, CT)])


@jax.jit
def kernel(hidden_states, Wg, w1, w2):
    pos, wts, endr = _router(hidden_states, Wg)
    posf = pos.reshape(A)
    wtsf = wts.reshape(A)
    endv = endr.reshape(E)
    xs, scale = _sc_dispatch()(hidden_states, posf, wtsf)
    ys = _gemm(endv, xs, scale.reshape(RMAX, 1), w1, w2)
    return _sc_combine()(ys, posf)
